# Initial kernel scaffold; baseline (speedup 1.0000x reference)
import jax
import jax.numpy as jnp
from jax.experimental import pallas as pl


def kernel(state, expert_id, w):
    raise NotImplementedError("write your pallas kernel here")


# speedup vs baseline: 1.4817x; 1.4817x over previous
import functools

import jax
import jax.numpy as jnp
from jax.experimental import pallas as pl
from jax.experimental.pallas import tpu as pltpu


def _matmul_kernel(expert_ref, x_ref, w_ref, o_ref):
    o_ref[...] = jax.lax.dot_general(
        x_ref[...], w_ref[0],
        dimension_numbers=(((1,), (1,)), ((), ())),
        preferred_element_type=jnp.float32,
    )


@functools.partial(jax.jit, static_argnames=())
def kernel(state, expert_id, w):
    M, K = state.shape
    E, N, K2 = w.shape
    BM, BN = 1024, 512
    expert = jnp.asarray(expert_id, dtype=jnp.int32).reshape((1,))

    grid = (M // BM, N // BN)
    out = pl.pallas_call(
        _matmul_kernel,
        grid_spec=pltpu.PrefetchScalarGridSpec(
            num_scalar_prefetch=1,
            grid=grid,
            in_specs=[
                pl.BlockSpec((BM, K), lambda i, j, e: (i, 0)),
                pl.BlockSpec((1, BN, K), lambda i, j, e: (e[0], j, 0)),
            ],
            out_specs=pl.BlockSpec((BM, BN), lambda i, j, e: (i, j)),
        ),
        out_shape=jax.ShapeDtypeStruct((M, N), jnp.float32),
        compiler_params=pltpu.CompilerParams(
            dimension_semantics=("parallel", "parallel"),
        ),
    )(expert, state, w)
    return out


# speedup vs baseline: 1.6725x; 1.1288x over previous
import functools

import jax
import jax.numpy as jnp
from jax.experimental import pallas as pl
from jax.experimental.pallas import tpu as pltpu


def _matmul_kernel(expert_ref, x_ref, w_ref, o_ref):
    o_ref[...] = jax.lax.dot_general(
        x_ref[...], w_ref[0],
        dimension_numbers=(((1,), (1,)), ((), ())),
        preferred_element_type=jnp.float32,
    )


@functools.partial(jax.jit, static_argnames=())
def kernel(state, expert_id, w):
    M, K = state.shape
    E, N, K2 = w.shape
    BM, BN = 2048, 512
    expert = jnp.asarray(expert_id, dtype=jnp.int32).reshape((1,))

    grid = (M // BM, N // BN)
    out = pl.pallas_call(
        _matmul_kernel,
        grid_spec=pltpu.PrefetchScalarGridSpec(
            num_scalar_prefetch=1,
            grid=grid,
            in_specs=[
                pl.BlockSpec((BM, K), lambda i, j, e: (i, 0)),
                pl.BlockSpec((1, BN, K), lambda i, j, e: (e[0], j, 0)),
            ],
            out_specs=pl.BlockSpec((BM, BN), lambda i, j, e: (i, j)),
        ),
        out_shape=jax.ShapeDtypeStruct((M, N), jnp.float32),
        compiler_params=pltpu.CompilerParams(
            dimension_semantics=("parallel", "parallel"),
        ),
    )(expert, state, w)
    return out
